# baseline (speedup 1.0000x reference)
import functools

import jax
import jax.numpy as jnp
from jax import lax
from jax.experimental import pallas as pl
from jax.experimental.pallas import tpu as pltpu
from jax.experimental.pallas import tpu_sc as plsc

IN_FEATURES = 1000000
OUT_FEATURES = 32
BATCH = 16384

NC, NS, L = 2, 16, 16
NW = NC * NS
B_PER_W = BATCH // NW
CHUNK = 64
NCHUNK = B_PER_W // CHUNK
NBUF = 2
SUPER = 128

_mesh = plsc.VectorSubcoreMesh(core_axis_name="c", subcore_axis_name="s")


@functools.partial(
    pl.kernel,
    mesh=_mesh,
    out_type=jax.ShapeDtypeStruct((BATCH, OUT_FEATURES), jnp.float32),
    scratch_types=[
        pltpu.VMEM((B_PER_W,), jnp.int32),
        pltpu.VMEM((NCHUNK, CHUNK), jnp.int32),
        pltpu.VMEM((NCHUNK, CHUNK), jnp.int32),
        pltpu.VMEM((NBUF, CHUNK, SUPER), jnp.float32),
        pltpu.VMEM((NBUF, CHUNK, OUT_FEATURES), jnp.float32),
        pltpu.VMEM((OUT_FEATURES,), jnp.float32),
        pltpu.SemaphoreType.DMA,
        pltpu.SemaphoreType.DMA,
    ],
)
def _sc_lookup(table_hbm, idx_hbm, bias_hbm, out_hbm,
               idx_v, tid_v, off_v, grp_v, outc_v, bias_v, sem0, sem1):
    wid = lax.axis_index("s") * NC + lax.axis_index("c")
    base = wid * B_PER_W
    sems = (sem0, sem1)

    pltpu.sync_copy(bias_hbm, bias_v)
    pltpu.sync_copy(idx_hbm.at[pl.ds(base, B_PER_W)], idx_v)

    for g in range(B_PER_W // L):
        v = idx_v[pl.ds(g * L, L)]
        j, r = divmod(g, CHUNK // L)
        tid_v[j, pl.ds(r * L, L)] = lax.shift_right_logical(v, 2)
        off_v[j, pl.ds(r * L, L)] = (v & 3) * OUT_FEATURES

    def fire(j):
        return pltpu.async_copy(
            table_hbm.at[tid_v.at[j]], grp_v.at[j % NBUF], sems[j % NBUF])

    bias_lo = bias_v[pl.ds(0, L)]
    bias_hi = bias_v[pl.ds(L, L)]
    pending = fire(0)
    for j in range(NCHUNK):
        pending.wait()
        if j + 1 < NCHUNK:
            pending = fire(j + 1)
        b = j % NBUF

        def group_body(g, _):
            off16 = off_v[j, pl.ds(g * L, L)]
            for l in range(L):
                i = g * L + l
                o = off16[l]
                outc_v[b, i, pl.ds(0, L)] = (
                    grp_v[b, i, pl.ds(o, L)] + bias_lo)
                outc_v[b, i, pl.ds(L, L)] = (
                    grp_v[b, i, pl.ds(o + L, L)] + bias_hi)
            return 0

        lax.fori_loop(0, CHUNK // L, group_body, 0)
        pltpu.sync_copy(outc_v.at[b],
                        out_hbm.at[pl.ds(base + j * CHUNK, CHUNK)])


def kernel(input, weight, bias):
    idx = input.astype(jnp.int32)
    table = weight.reshape(IN_FEATURES // 4, 4 * OUT_FEATURES)
    return _sc_lookup(table, idx, bias)


# speedup vs baseline: 1.6672x; 1.6672x over previous
import functools

import jax
import jax.numpy as jnp
from jax import lax
from jax.experimental import pallas as pl
from jax.experimental.pallas import tpu as pltpu
from jax.experimental.pallas import tpu_sc as plsc

IN_FEATURES = 1000000
OUT_FEATURES = 32
BATCH = 16384

NC, NS, L = 2, 16, 16
NW = NC * NS
B_PER_W = BATCH // NW
WAVE = 128
NWAVE = B_PER_W // WAVE
NSEM = 8

_mesh = plsc.VectorSubcoreMesh(core_axis_name="c", subcore_axis_name="s")


@functools.partial(
    pl.kernel,
    mesh=_mesh,
    out_type=jax.ShapeDtypeStruct((BATCH, OUT_FEATURES), jnp.float32),
    scratch_types=[
        pltpu.VMEM((B_PER_W,), jnp.int32),
        pltpu.VMEM((B_PER_W, OUT_FEATURES), jnp.float32),
        pltpu.VMEM((OUT_FEATURES,), jnp.float32),
    ] + [pltpu.SemaphoreType.DMA] * NSEM,
)
def _sc_lookup(table_hbm, idx_hbm, bias_hbm, out_hbm,
               idx_v, rows_v, bias_v, *sems):
    wid = lax.axis_index("s") * NC + lax.axis_index("c")
    base = wid * B_PER_W

    pltpu.sync_copy(bias_hbm, bias_v)
    pltpu.sync_copy(idx_hbm.at[pl.ds(base, B_PER_W)], idx_v)
    bias_lo = bias_v[pl.ds(0, L)]
    bias_hi = bias_v[pl.ds(L, L)]

    def fire(w):
        def enqueue(g, _):
            r16 = idx_v[pl.ds(w * WAVE + g * L, L)]
            for l in range(L):
                i = w * WAVE + g * L + l
                pltpu.async_copy(table_hbm.at[r16[l]], rows_v.at[i],
                                 sems[l % NSEM])
            return 0

        lax.fori_loop(0, WAVE // L, enqueue, 0)

    def drain(w):
        for k in range(NSEM):
            pltpu.make_async_copy(
                out_hbm.at[pl.ds(base, WAVE // NSEM)],
                rows_v.at[pl.ds(0, WAVE // NSEM)],
                sems[k],
            ).wait()

    def add_bias(w):
        def body(i, _):
            rows_v[i, pl.ds(0, L)] += bias_lo
            rows_v[i, pl.ds(L, L)] += bias_hi
            return 0

        lax.fori_loop(w * WAVE, (w + 1) * WAVE, body, 0, unroll=4)

    fire(0)
    for w in range(NWAVE):
        if w + 1 < NWAVE:
            fire(w + 1)
        drain(w)
        add_bias(w)
    pltpu.sync_copy(rows_v, out_hbm.at[pl.ds(base, B_PER_W)])


def kernel(input, weight, bias):
    idx = input.astype(jnp.int32)
    return _sc_lookup(weight, idx, bias)
